# baseline (speedup 1.0000x reference)
import functools

import jax
import jax.numpy as jnp
from jax import lax
from jax.experimental import pallas as pl
from jax.experimental.pallas import tpu as pltpu
from jax.experimental.pallas import tpu_sc as plsc

N_WORD = 5000
D = 128
N_CAT = 50
L = 20
B = 1024
K = 16
XPAD = 24

NC = 2
NS = 16
NW = NC * NS
B_PER_W = B // NW

_GATHER_DN = lax.GatherDimensionNumbers(
    offset_dims=(), collapsed_slice_dims=(0,), start_index_map=(0,))


def _gather_vec(vec, idx):
    return lax.gather(vec, idx.reshape(16, 1), _GATHER_DN, (1,),
                      mode=lax.GatherScatterMode.PROMISE_IN_BOUNDS)


def _bcast_lane(vec, k):
    return _gather_vec(vec, jnp.full((16,), k, jnp.int32))


def _ln_body(emb_ref, g_ref, b_ref, o_ref):
    h = emb_ref[...]
    mu = jnp.mean(h, axis=1, keepdims=True)
    var = jnp.mean((h - mu) ** 2, axis=1, keepdims=True)
    o_ref[...] = (h - mu) * lax.rsqrt(var + 1e-5) * g_ref[...] + b_ref[...]


def _layer_norm_table(emb_table, gamma, beta):
    return pl.pallas_call(
        _ln_body,
        out_shape=jax.ShapeDtypeStruct((N_WORD, D), jnp.float32),
    )(emb_table, gamma.reshape(1, D), beta.reshape(1, D))


def _fc_body(h_ref, w_ref, b_ref, o_ref):
    o_ref[...] = (
        lax.dot_general(
            h_ref[...], w_ref[...],
            dimension_numbers=(((1,), (1,)), ((), ())),
            preferred_element_type=jnp.float32,
        )
        + b_ref[...]
    )


def _classify(hsum, fc_W, fc_b):
    return pl.pallas_call(
        _fc_body,
        out_shape=jax.ShapeDtypeStruct((B, N_CAT), jnp.float32),
    )(hsum, fc_W, fc_b.reshape(1, N_CAT))


def _sq_body(in_hbm, o_hbm, sem):
    n = o_hbm.shape[0]
    c = min(n, 3056 * 128)
    cps = []
    lo = 0
    while lo < n:
        sz = min(c, n - lo)
        cps.append(pltpu.make_async_copy(
            in_hbm.at[0].at[pl.ds(lo, sz)], o_hbm.at[pl.ds(lo, sz)], sem))
        lo += sz
    for cp in cps:
        cp.start()
    for cp in cps:
        cp.wait()


def _flatten_col(t2):
    n = t2.shape[0]
    npad = (-n) % 128
    tp = jnp.pad(t2, ((0, npad), (0, 0)))
    return pl.pallas_call(
        _sq_body,
        in_specs=[pl.BlockSpec(memory_space=pl.ANY)],
        out_specs=pl.BlockSpec(memory_space=pl.ANY),
        out_shape=jax.ShapeDtypeStruct((n + npad,), jnp.float32),
        scratch_shapes=[pltpu.SemaphoreType.DMA],
    )(tp.T)


def _sc_body(xp_hbm, nb_hbm, we_hbm, lnemb_hbm, edge_hbm, eta_hbm, out_hbm,
             xv, nbv, wev, ebuf, cbuf, wbuf, etav, hbuf, sem_i, sem_d):
    wid = lax.axis_index("s") * NC + lax.axis_index("c")

    def do_sentence(i, carry):
        b = wid * B_PER_W + i

        c_x = pltpu.make_async_copy(xp_hbm.at[pl.ds(b * XPAD, XPAD)], xv, sem_i)
        c_nb = pltpu.make_async_copy(nb_hbm.at[pl.ds(b * L * K, L * K)], nbv, sem_i)
        c_we = pltpu.make_async_copy(we_hbm.at[pl.ds(b * L * K, L * K)], wev, sem_i)
        c_x.start(); c_nb.start(); c_we.start()
        c_x.wait(); c_nb.wait(); c_we.wait()

        gathers = [
            pltpu.make_async_copy(
                lnemb_hbm.at[nbv.at[pl.ds(0, 128)]], ebuf.at[pl.ds(0, 128)], sem_d),
            pltpu.make_async_copy(
                lnemb_hbm.at[nbv.at[pl.ds(128, 128)]], ebuf.at[pl.ds(128, 128)], sem_d),
            pltpu.make_async_copy(
                lnemb_hbm.at[nbv.at[pl.ds(256, 64)]], ebuf.at[pl.ds(256, 64)], sem_d),
            pltpu.make_async_copy(
                edge_hbm.at[wev.at[pl.ds(0, 128)]], wbuf.at[pl.ds(0, 128)], sem_d),
            pltpu.make_async_copy(
                edge_hbm.at[wev.at[pl.ds(128, 128)]], wbuf.at[pl.ds(128, 128)], sem_d),
            pltpu.make_async_copy(
                edge_hbm.at[wev.at[pl.ds(256, 64)]], wbuf.at[pl.ds(256, 64)], sem_d),
            pltpu.make_async_copy(
                lnemb_hbm.at[xv.at[pl.ds(0, L)]], cbuf, sem_d),
            pltpu.make_async_copy(
                eta_hbm.at[xv.at[pl.ds(0, L)]], etav.at[pl.ds(0, L)], sem_d),
        ]
        for g in gathers:
            g.start()
        for g in gathers:
            g.wait()

        def token(l, acc):
            row0 = l * K
            wv = wbuf[pl.ds(row0, K)]
            ev = etav[pl.ds(l, 16)]
            eta_b = _bcast_lane(ev, 0)
            w_b = _bcast_lane(wv, 0)
            msg = [w_b * ebuf[row0, pl.ds(c * 16, 16)] for c in range(8)]
            for k in range(1, K):
                w_b = _bcast_lane(wv, k)
                for c in range(8):
                    msg[c] = jnp.maximum(
                        msg[c], w_b * ebuf[row0 + k, pl.ds(c * 16, 16)])
            new = []
            for c in range(8):
                cen = cbuf[l, pl.ds(c * 16, 16)]
                h = (1.0 - eta_b) * msg[c] + eta_b * cen
                new.append(acc[c] + h)
            return tuple(new)

        acc0 = tuple(jnp.zeros((16,), jnp.float32) for _ in range(8))
        acc = lax.fori_loop(0, L, token, acc0)

        for c in range(8):
            hbuf[pl.ds(c * 16, 16)] = acc[c]
        pltpu.sync_copy(hbuf, out_hbm.at[pl.ds(b * D, D)])
        return carry

    lax.fori_loop(0, B_PER_W, do_sentence, 0)


def _sc_call(xp, nb, we, lnemb, edge_flat, eta_table):
    mesh = plsc.VectorSubcoreMesh(core_axis_name="c", subcore_axis_name="s")
    f = functools.partial(
        pl.kernel,
        out_type=jax.ShapeDtypeStruct((B * D,), jnp.float32),
        mesh=mesh,
        scratch_types=[
            pltpu.VMEM((XPAD,), jnp.int32),
            pltpu.VMEM((L * K,), jnp.int32),
            pltpu.VMEM((L * K,), jnp.int32),
            pltpu.VMEM((L * K, D), jnp.float32),
            pltpu.VMEM((L, D), jnp.float32),
            pltpu.VMEM((384,), jnp.float32),
            pltpu.VMEM((128,), jnp.float32),
            pltpu.VMEM((D,), jnp.float32),
            pltpu.SemaphoreType.DMA,
            pltpu.SemaphoreType.DMA,
        ],
    )(_sc_body)
    return f(xp, nb, we, lnemb, edge_flat, eta_table)


def kernel(x, nb_x, w_edge, emb_table, edge_table, eta_table,
           ln_gamma, ln_beta, fc_W, fc_b):
    x = x.astype(jnp.int32)
    nb_x = nb_x.astype(jnp.int32)
    w_edge = w_edge.astype(jnp.int32)

    lnemb = _layer_norm_table(emb_table, ln_gamma, ln_beta)

    xp = jnp.pad(x, ((0, 0), (0, XPAD - L))).reshape(-1)
    nb = nb_x.reshape(-1)
    we = w_edge.reshape(-1)

    edge_flat = _flatten_col(edge_table)

    hsum = _sc_call(xp, nb, we, lnemb, edge_flat,
                    _flatten_col(eta_table)).reshape(B, D)
    return _classify(hsum, fc_W, fc_b)


# speedup vs baseline: 14.8036x; 14.8036x over previous
import functools

import jax
import jax.numpy as jnp
from jax import lax
from jax.experimental import pallas as pl
from jax.experimental.pallas import tpu as pltpu
from jax.experimental.pallas import tpu_sc as plsc

N_WORD = 5000
D = 128
N_CAT = 50
L = 20
B = 1024
K = 16
XPAD = 24

NC = 2
NS = 16
NW = NC * NS
B_PER_W = B // NW

_GATHER_DN = lax.GatherDimensionNumbers(
    offset_dims=(), collapsed_slice_dims=(0,), start_index_map=(0,))


def _gather_vec(vec, idx):
    return lax.gather(vec, idx.reshape(16, 1), _GATHER_DN, (1,),
                      mode=lax.GatherScatterMode.PROMISE_IN_BOUNDS)


def _bcast_lane(vec, k):
    return _gather_vec(vec, jnp.full((16,), k, jnp.int32))


def _ln_body(emb_ref, g_ref, b_ref, o_ref):
    h = emb_ref[...]
    mu = jnp.mean(h, axis=1, keepdims=True)
    var = jnp.mean((h - mu) ** 2, axis=1, keepdims=True)
    o_ref[...] = (h - mu) * lax.rsqrt(var + 1e-5) * g_ref[...] + b_ref[...]


def _layer_norm_table(emb_table, gamma, beta):
    return pl.pallas_call(
        _ln_body,
        out_shape=jax.ShapeDtypeStruct((N_WORD, D), jnp.float32),
    )(emb_table, gamma.reshape(1, D), beta.reshape(1, D))


def _fc_body(h_ref, w_ref, b_ref, o_ref):
    o_ref[...] = (
        lax.dot_general(
            h_ref[...], w_ref[...],
            dimension_numbers=(((1,), (1,)), ((), ())),
            preferred_element_type=jnp.float32,
        )
        + b_ref[...]
    )


def _classify(hsum, fc_W, fc_b):
    return pl.pallas_call(
        _fc_body,
        out_shape=jax.ShapeDtypeStruct((B, N_CAT), jnp.float32),
    )(hsum, fc_W, fc_b.reshape(1, N_CAT))


def _sc_body(xp_hbm, nb_hbm, we_hbm, lnemb_hbm, edge_hbm, eta_hbm, out_hbm,
             xv, nbv, wev, ebuf, cbuf, wbuf, etav, hbuf, sem_i, sem_d):
    wid = lax.axis_index("s") * NC + lax.axis_index("c")

    edge_flat = edge_hbm.at[0]
    eta_flat = eta_hbm.at[0]

    def do_sentence(i, carry):
        b = wid * B_PER_W + i

        c_x = pltpu.make_async_copy(xp_hbm.at[pl.ds(b * XPAD, XPAD)], xv, sem_i)
        c_nb = pltpu.make_async_copy(nb_hbm.at[pl.ds(b * L * K, L * K)], nbv, sem_i)
        c_we = pltpu.make_async_copy(we_hbm.at[pl.ds(b * L * K, L * K)], wev, sem_i)
        c_x.start(); c_nb.start(); c_we.start()
        c_x.wait(); c_nb.wait(); c_we.wait()

        gathers = [
            pltpu.make_async_copy(
                lnemb_hbm.at[nbv.at[pl.ds(0, 128)]], ebuf.at[pl.ds(0, 128)], sem_d),
            pltpu.make_async_copy(
                lnemb_hbm.at[nbv.at[pl.ds(128, 128)]], ebuf.at[pl.ds(128, 128)], sem_d),
            pltpu.make_async_copy(
                lnemb_hbm.at[nbv.at[pl.ds(256, 64)]], ebuf.at[pl.ds(256, 64)], sem_d),
            pltpu.make_async_copy(
                edge_flat.at[wev.at[pl.ds(0, 128)]], wbuf.at[pl.ds(0, 128)], sem_d),
            pltpu.make_async_copy(
                edge_flat.at[wev.at[pl.ds(128, 128)]], wbuf.at[pl.ds(128, 128)], sem_d),
            pltpu.make_async_copy(
                edge_flat.at[wev.at[pl.ds(256, 64)]], wbuf.at[pl.ds(256, 64)], sem_d),
            pltpu.make_async_copy(
                lnemb_hbm.at[xv.at[pl.ds(0, L)]], cbuf, sem_d),
            pltpu.make_async_copy(
                eta_flat.at[xv.at[pl.ds(0, L)]], etav.at[pl.ds(0, L)], sem_d),
        ]
        for g in gathers:
            g.start()
        for g in gathers:
            g.wait()

        def token(l, acc):
            row0 = l * K
            wv = wbuf[pl.ds(row0, K)]
            ev = etav[pl.ds(l, 16)]
            eta_b = _bcast_lane(ev, 0)
            w_b = _bcast_lane(wv, 0)
            msg = [w_b * ebuf[row0, pl.ds(c * 16, 16)] for c in range(8)]
            for k in range(1, K):
                w_b = _bcast_lane(wv, k)
                for c in range(8):
                    msg[c] = jnp.maximum(
                        msg[c], w_b * ebuf[row0 + k, pl.ds(c * 16, 16)])
            new = []
            for c in range(8):
                cen = cbuf[l, pl.ds(c * 16, 16)]
                h = (1.0 - eta_b) * msg[c] + eta_b * cen
                new.append(acc[c] + h)
            return tuple(new)

        acc0 = tuple(jnp.zeros((16,), jnp.float32) for _ in range(8))
        acc = lax.fori_loop(0, L, token, acc0)

        for c in range(8):
            hbuf[pl.ds(c * 16, 16)] = acc[c]
        pltpu.sync_copy(hbuf, out_hbm.at[pl.ds(b * D, D)])
        return carry

    lax.fori_loop(0, B_PER_W, do_sentence, 0)


def _sc_call(xp, nb, we, lnemb, edge_flat, eta_table):
    mesh = plsc.VectorSubcoreMesh(core_axis_name="c", subcore_axis_name="s")
    f = functools.partial(
        pl.kernel,
        out_type=jax.ShapeDtypeStruct((B * D,), jnp.float32),
        mesh=mesh,
        scratch_types=[
            pltpu.VMEM((XPAD,), jnp.int32),
            pltpu.VMEM((L * K,), jnp.int32),
            pltpu.VMEM((L * K,), jnp.int32),
            pltpu.VMEM((L * K, D), jnp.float32),
            pltpu.VMEM((L, D), jnp.float32),
            pltpu.VMEM((384,), jnp.float32),
            pltpu.VMEM((128,), jnp.float32),
            pltpu.VMEM((D,), jnp.float32),
            pltpu.SemaphoreType.DMA,
            pltpu.SemaphoreType.DMA,
        ],
    )(_sc_body)
    return f(xp, nb, we, lnemb, edge_flat, eta_table)


def kernel(x, nb_x, w_edge, emb_table, edge_table, eta_table,
           ln_gamma, ln_beta, fc_W, fc_b):
    x = x.astype(jnp.int32)
    nb_x = nb_x.astype(jnp.int32)
    w_edge = w_edge.astype(jnp.int32)

    lnemb = _layer_norm_table(emb_table, ln_gamma, ln_beta)

    xp = jnp.pad(x, ((0, 0), (0, XPAD - L))).reshape(-1)
    nb = nb_x.reshape(-1)
    we = w_edge.reshape(-1)

    hsum = _sc_call(xp, nb, we, lnemb, edge_table.T,
                    eta_table.T).reshape(B, D)
    return _classify(hsum, fc_W, fc_b)


# speedup vs baseline: 22.2169x; 1.5008x over previous
import functools

import jax
import jax.numpy as jnp
from jax import lax
from jax.experimental import pallas as pl
from jax.experimental.pallas import tpu as pltpu
from jax.experimental.pallas import tpu_sc as plsc

N_WORD = 5000
D = 128
N_CAT = 50
L = 20
B = 1024
K = 16
XPAD = 24

NC = 2
NS = 16
NW = NC * NS
B_PER_W = B // NW

_GATHER_DN = lax.GatherDimensionNumbers(
    offset_dims=(), collapsed_slice_dims=(0,), start_index_map=(0,))


def _gather_vec(vec, idx):
    return lax.gather(vec, idx.reshape(16, 1), _GATHER_DN, (1,),
                      mode=lax.GatherScatterMode.PROMISE_IN_BOUNDS)


def _bcast_lane(vec, k):
    return _gather_vec(vec, jnp.full((16,), k, jnp.int32))


def _ln_body(emb_ref, g_ref, b_ref, o_ref):
    h = emb_ref[...]
    mu = jnp.mean(h, axis=1, keepdims=True)
    var = jnp.mean((h - mu) ** 2, axis=1, keepdims=True)
    o_ref[...] = (h - mu) * lax.rsqrt(var + 1e-5) * g_ref[...] + b_ref[...]


def _layer_norm_table(emb_table, gamma, beta):
    return pl.pallas_call(
        _ln_body,
        out_shape=jax.ShapeDtypeStruct((N_WORD, D), jnp.float32),
    )(emb_table, gamma.reshape(1, D), beta.reshape(1, D))


def _fc_body(h_ref, w_ref, b_ref, o_ref):
    o_ref[...] = (
        lax.dot_general(
            h_ref[...], w_ref[...],
            dimension_numbers=(((1,), (1,)), ((), ())),
            preferred_element_type=jnp.float32,
        )
        + b_ref[...]
    )


def _classify(hsum, fc_W, fc_b):
    return pl.pallas_call(
        _fc_body,
        out_shape=jax.ShapeDtypeStruct((B, N_CAT), jnp.float32),
    )(hsum, fc_W, fc_b.reshape(1, N_CAT))


def _sc_body(xp_hbm, nb_hbm, we_hbm, lnemb_hbm, edge_hbm, eta_hbm, out_hbm,
             xv0, nbv0, wev0, xv1, nbv1, wev1,
             ebuf0, cbuf0, wbuf0, etav0, ebuf1, cbuf1, wbuf1, etav1, hbuf,
             si0, si1, sd0, sd1):
    wid = lax.axis_index("s") * NC + lax.axis_index("c")
    edge_flat = edge_hbm.at[0]
    eta_flat = eta_hbm.at[0]

    IDX0 = (xv0, nbv0, wev0)
    IDX1 = (xv1, nbv1, wev1)
    DAT0 = (ebuf0, cbuf0, wbuf0, etav0)
    DAT1 = (ebuf1, cbuf1, wbuf1, etav1)

    def idx_copies(bufs, sem, b):
        xv, nbv, wev = bufs
        return [
            pltpu.make_async_copy(xp_hbm.at[pl.ds(b * XPAD, XPAD)], xv, sem),
            pltpu.make_async_copy(nb_hbm.at[pl.ds(b * L * K, L * K)], nbv, sem),
            pltpu.make_async_copy(we_hbm.at[pl.ds(b * L * K, L * K)], wev, sem),
        ]

    def gather_copies(ibufs, dbufs, sem):
        xv, nbv, wev = ibufs
        ebuf, cbuf, wbuf, etav = dbufs
        return [
            pltpu.make_async_copy(
                lnemb_hbm.at[nbv.at[pl.ds(0, 128)]], ebuf.at[pl.ds(0, 128)], sem),
            pltpu.make_async_copy(
                lnemb_hbm.at[nbv.at[pl.ds(128, 128)]], ebuf.at[pl.ds(128, 128)], sem),
            pltpu.make_async_copy(
                lnemb_hbm.at[nbv.at[pl.ds(256, 64)]], ebuf.at[pl.ds(256, 64)], sem),
            pltpu.make_async_copy(
                edge_flat.at[wev.at[pl.ds(0, 128)]], wbuf.at[pl.ds(0, 128)], sem),
            pltpu.make_async_copy(
                edge_flat.at[wev.at[pl.ds(128, 128)]], wbuf.at[pl.ds(128, 128)], sem),
            pltpu.make_async_copy(
                edge_flat.at[wev.at[pl.ds(256, 64)]], wbuf.at[pl.ds(256, 64)], sem),
            pltpu.make_async_copy(
                lnemb_hbm.at[xv.at[pl.ds(0, L)]], cbuf, sem),
            pltpu.make_async_copy(
                eta_flat.at[xv.at[pl.ds(0, L)]], etav.at[pl.ds(0, L)], sem),
        ]

    def fire(cps):
        for cp in cps:
            cp.start()

    def drain(cps):
        for cp in cps:
            cp.wait()

    def compute(dbufs, b):
        ebuf, cbuf, wbuf, etav = dbufs

        def token(l, acc):
            row0 = l * K
            wv = wbuf[pl.ds(row0, K)]
            ev = etav[pl.ds(l, 16)]
            eta_b = _bcast_lane(ev, 0)
            w_b = _bcast_lane(wv, 0)
            msg = [w_b * ebuf[row0, pl.ds(c * 16, 16)] for c in range(8)]
            for k in range(1, K):
                w_b = _bcast_lane(wv, k)
                for c in range(8):
                    msg[c] = jnp.maximum(
                        msg[c], w_b * ebuf[row0 + k, pl.ds(c * 16, 16)])
            new = []
            for c in range(8):
                cen = cbuf[l, pl.ds(c * 16, 16)]
                h = (1.0 - eta_b) * msg[c] + eta_b * cen
                new.append(acc[c] + h)
            return tuple(new)

        acc = lax.fori_loop(0, L, token,
                            tuple(jnp.zeros((16,), jnp.float32) for _ in range(8)))
        for c in range(8):
            hbuf[pl.ds(c * 16, 16)] = acc[c]
        pltpu.sync_copy(hbuf, out_hbm.at[pl.ds(b * D, D)])

    base = wid * B_PER_W
    fire(idx_copies(IDX0, si0, base))
    drain(idx_copies(IDX0, si0, base))
    fire(gather_copies(IDX0, DAT0, sd0))
    fire(idx_copies(IDX1, si1, base + 1))

    def pair(p, carry):
        a = base + 2 * p
        b = a + 1
        drain(idx_copies(IDX1, si1, b))
        fire(gather_copies(IDX1, DAT1, sd1))
        drain(gather_copies(IDX0, DAT0, sd0))

        @pl.when(p < B_PER_W // 2 - 1)
        def _():
            fire(idx_copies(IDX0, si0, a + 2))

        compute(DAT0, a)

        drain(gather_copies(IDX1, DAT1, sd1))

        @pl.when(p < B_PER_W // 2 - 1)
        def _():
            fire(idx_copies(IDX1, si1, b + 2))
            drain(idx_copies(IDX0, si0, a + 2))
            fire(gather_copies(IDX0, DAT0, sd0))

        compute(DAT1, b)
        return carry

    lax.fori_loop(0, B_PER_W // 2, pair, 0)


def _sc_call(xp, nb, we, lnemb, edge_flat, eta_table):
    mesh = plsc.VectorSubcoreMesh(core_axis_name="c", subcore_axis_name="s")
    ibufs = [
        pltpu.VMEM((XPAD,), jnp.int32),
        pltpu.VMEM((L * K,), jnp.int32),
        pltpu.VMEM((L * K,), jnp.int32),
    ]
    dbufs = [
        pltpu.VMEM((L * K, D), jnp.float32),
        pltpu.VMEM((L, D), jnp.float32),
        pltpu.VMEM((384,), jnp.float32),
        pltpu.VMEM((128,), jnp.float32),
    ]
    f = functools.partial(
        pl.kernel,
        out_type=jax.ShapeDtypeStruct((B * D,), jnp.float32),
        mesh=mesh,
        scratch_types=ibufs + ibufs + dbufs + dbufs + [
            pltpu.VMEM((D,), jnp.float32),
            pltpu.SemaphoreType.DMA,
            pltpu.SemaphoreType.DMA,
            pltpu.SemaphoreType.DMA,
            pltpu.SemaphoreType.DMA,
        ],
    )(_sc_body)
    return f(xp, nb, we, lnemb, edge_flat, eta_table)


def kernel(x, nb_x, w_edge, emb_table, edge_table, eta_table,
           ln_gamma, ln_beta, fc_W, fc_b):
    x = x.astype(jnp.int32)
    nb_x = nb_x.astype(jnp.int32)
    w_edge = w_edge.astype(jnp.int32)

    lnemb = _layer_norm_table(emb_table, ln_gamma, ln_beta)

    xp = jnp.pad(x, ((0, 0), (0, XPAD - L))).reshape(-1)
    nb = nb_x.reshape(-1)
    we = w_edge.reshape(-1)

    hsum = _sc_call(xp, nb, we, lnemb, edge_table.T,
                    eta_table.T).reshape(B, D)
    return _classify(hsum, fc_W, fc_b)
